# Initial kernel scaffold; baseline (speedup 1.0000x reference)
import functools

import jax
import jax.numpy as jnp
from jax import lax
from jax.experimental import pallas as pl
from jax.experimental.pallas import tpu as pltpu
from jax.experimental.pallas import tpu_sc as plsc

B = 16384
L = 20
D = 64

NC = 2
NS = 16
NW = NC * NS
BPW = B // NW
CH = 64
NCHUNK = BPW // CH
ROWS = CH * L
NIDX = ROWS // 128
UIDX = BPW // 128


def _sc_gather(title2d, user2d, domain2d, dstidx, word_emb, user_emb,
               domain_emb):
  mesh = plsc.VectorSubcoreMesh(core_axis_name="c", subcore_axis_name="s")

  @functools.partial(
      pl.kernel,
      out_type=(
          jax.ShapeDtypeStruct((B, D), jnp.float32),
          jax.ShapeDtypeStruct((B, D), jnp.float32),
          jax.ShapeDtypeStruct((B, D), jnp.float32),
      ),
      mesh=mesh,
      scratch_types=[
          pltpu.VMEM((NIDX, 128), jnp.int32),
          pltpu.VMEM((UIDX, 128), jnp.int32),
          pltpu.VMEM((NIDX, 128), jnp.int32),
          pltpu.VMEM((ROWS, D), jnp.float32),
          pltpu.VMEM((CH, D), jnp.float32),
          pltpu.VMEM_SHARED((NS * CH, D), jnp.float32),
          pltpu.SemaphoreType.DMA,
      ],
  )
  def k(title_hbm, user_hbm, domain_hbm, dsti_hbm, wemb, uemb, demb,
        ts_out, uv_out, dv_out,
        tidx_v, uidx_v, dsti_v, rows_v, zeros_v, acc_sh, sem):
    cid = lax.axis_index("c")
    sid = lax.axis_index("s")
    wid = sid * NC + cid
    base = wid * BPW

    pltpu.sync_copy(dsti_hbm.at[:], dsti_v)
    off = sid * CH
    for i in range(NIDX):
      for j in range(8):
        s = pl.ds(j * 16, 16)
        dsti_v[i, s] = dsti_v[i, s] + off

    zero16 = jnp.zeros((16,), jnp.float32)
    for r in range(CH):
      for j in range(D // 16):
        zeros_v[r, pl.ds(j * 16, 16)] = zero16

    pltpu.sync_copy(user_hbm.at[pl.ds(wid * UIDX, UIDX)], uidx_v)
    for i in range(UIDX):
      pltpu.async_copy(uemb.at[uidx_v.at[i]],
                       rows_v.at[pl.ds(i * 128, 128)], sem)
    for i in range(UIDX):
      pltpu.make_async_copy(uemb.at[uidx_v.at[i]],
                            rows_v.at[pl.ds(i * 128, 128)], sem).wait()
    pltpu.sync_copy(rows_v.at[pl.ds(0, BPW)], uv_out.at[pl.ds(base, BPW)])

    pltpu.sync_copy(domain_hbm.at[pl.ds(wid * UIDX, UIDX)], uidx_v)
    for i in range(UIDX):
      pltpu.async_copy(demb.at[uidx_v.at[i]],
                       rows_v.at[pl.ds(i * 128, 128)], sem)
    for i in range(UIDX):
      pltpu.make_async_copy(demb.at[uidx_v.at[i]],
                            rows_v.at[pl.ds(i * 128, 128)], sem).wait()
    pltpu.sync_copy(rows_v.at[pl.ds(0, BPW)], dv_out.at[pl.ds(base, BPW)])

    def chunk_body(c, carry):
      pltpu.sync_copy(
          title_hbm.at[pl.ds(wid * (NIDX * NCHUNK) + c * NIDX, NIDX)], tidx_v)
      for i in range(NIDX):
        pltpu.async_copy(wemb.at[tidx_v.at[i]],
                         rows_v.at[pl.ds(i * 128, 128)], sem)
      pltpu.sync_copy(zeros_v, acc_sh.at[pl.ds(off, CH)])
      for i in range(NIDX):
        pltpu.make_async_copy(wemb.at[tidx_v.at[i]],
                              rows_v.at[pl.ds(i * 128, 128)], sem).wait()
      for i in range(NIDX):
        pltpu.sync_copy(rows_v.at[pl.ds(i * 128, 128)],
                        acc_sh.at[dsti_v.at[i]], add=True)
      pltpu.sync_copy(acc_sh.at[pl.ds(off, CH)],
                      ts_out.at[pl.ds(base + c * CH, CH)])
      return carry

    lax.fori_loop(0, NCHUNK, chunk_body, 0)

  return k(title2d, user2d, domain2d, dstidx, word_emb, user_emb, domain_emb)


def _mlp(ts, uv, dv, w1t, w1u, w1d, b1, w2, b2):
  BLK = 1024

  def body(ts_ref, uv_ref, dv_ref, w1t_ref, w1u_ref, w1d_ref, b1_ref, w2_ref,
           b2_ref, out_ref):
    acc = lax.dot_general(ts_ref[...], w1t_ref[...], (((1,), (0,)), ((), ())),
                          preferred_element_type=jnp.float32)
    acc += lax.dot_general(uv_ref[...], w1u_ref[...], (((1,), (0,)), ((), ())),
                           preferred_element_type=jnp.float32)
    acc += lax.dot_general(dv_ref[...], w1d_ref[...], (((1,), (0,)), ((), ())),
                           preferred_element_type=jnp.float32)
    h = jnp.maximum(acc + b1_ref[...][None, :], 0.0)
    out_ref[...] = jnp.sum(h * w2_ref[...][None, :], axis=1,
                           keepdims=True) + b2_ref[0]

  grid = B // BLK
  return pl.pallas_call(
      body,
      grid=(grid,),
      in_specs=[
          pl.BlockSpec((BLK, D), lambda i: (i, 0)),
          pl.BlockSpec((BLK, D), lambda i: (i, 0)),
          pl.BlockSpec((BLK, D), lambda i: (i, 0)),
          pl.BlockSpec((D, 128), lambda i: (0, 0)),
          pl.BlockSpec((D, 128), lambda i: (0, 0)),
          pl.BlockSpec((D, 128), lambda i: (0, 0)),
          pl.BlockSpec((128,), lambda i: (0,)),
          pl.BlockSpec((128,), lambda i: (0,)),
          pl.BlockSpec(memory_space=pltpu.SMEM),
      ],
      out_specs=pl.BlockSpec((BLK, 1), lambda i: (i, 0)),
      out_shape=jax.ShapeDtypeStruct((B, 1), jnp.float32),
  )(ts, uv, dv, w1t, w1u, w1d, b1, w2, b2)


def kernel(title, user, domain, word_emb, user_emb, domain_emb, fc1_w, fc1_b,
           fc2_w, fc2_b):
  title2d = title.astype(jnp.int32).reshape(B * L // 128, 128)
  user2d = user.astype(jnp.int32).reshape(B // 128, 128)
  domain2d = domain.astype(jnp.int32).reshape(B // 128, 128)
  dstidx = (jnp.arange(ROWS, dtype=jnp.int32) // L).reshape(NIDX, 128)

  ts, uv, dv = _sc_gather(title2d, user2d, domain2d, dstidx, word_emb,
                          user_emb, domain_emb)

  w1t = fc1_w[:, :D].T * (1.0 / L)
  w1u = fc1_w[:, D:2 * D].T
  w1d = fc1_w[:, 2 * D:].T
  w2 = fc2_w[0]
  out = _mlp(ts, uv, dv, w1t, w1u, w1d, fc1_b, w2, fc2_b)
  return out[:, 0]


# speedup vs baseline: 5.0322x; 5.0322x over previous
import functools

import jax
import jax.numpy as jnp
from jax import lax
from jax.experimental import pallas as pl
from jax.experimental.pallas import tpu as pltpu
from jax.experimental.pallas import tpu_sc as plsc

B = 16384
L = 20
D = 64

NC = 2
NS = 16
NW = NC * NS
BPW = B // NW
CH = 64
NCHUNK = BPW // CH
ROWS = CH * L
NIDX = ROWS // 128
UIDX = BPW // 128


def _sc_gather(title2d, user2d, domain2d, dstidx, word_emb, user_emb,
               domain_emb):
  mesh = plsc.VectorSubcoreMesh(core_axis_name="c", subcore_axis_name="s",
                                num_cores=NC, num_subcores=NS)

  @functools.partial(
      pl.kernel,
      out_type=(
          jax.ShapeDtypeStruct((B, D), jnp.float32),
          jax.ShapeDtypeStruct((B, D), jnp.float32),
          jax.ShapeDtypeStruct((B, D), jnp.float32),
      ),
      mesh=mesh,
      scratch_types=[
          pltpu.VMEM((ROWS,), jnp.int32),
          pltpu.VMEM((BPW,), jnp.int32),
          pltpu.VMEM((NIDX, 128), jnp.int32),
          pltpu.VMEM((ROWS, D), jnp.float32),
          pltpu.VMEM((CH, D), jnp.float32),
          pltpu.VMEM_SHARED((NS * CH, D), jnp.float32),
          pltpu.SemaphoreType.DMA,
      ],
      compiler_params=pltpu.CompilerParams(use_tc_tiling_on_sc=False),
  )
  def k(title_hbm, user_hbm, domain_hbm, dsti_hbm, wemb, uemb, demb,
        ts_out, uv_out, dv_out,
        tidx_v, uidx_v, dsti_v, rows_v, zeros_v, acc_sh, sem):
    cid = lax.axis_index("c")
    sid = lax.axis_index("s")
    wid = sid * NC + cid
    base = wid * BPW

    pltpu.sync_copy(dsti_hbm.at[:], dsti_v)
    off = sid * CH
    for i in range(NIDX):
      for j in range(8):
        s = pl.ds(j * 16, 16)
        dsti_v[i, s] = dsti_v[i, s] + off

    zero16 = jnp.zeros((16,), jnp.float32)
    for r in range(CH):
      for j in range(D // 16):
        zeros_v[r, pl.ds(j * 16, 16)] = zero16

    pltpu.sync_copy(user_hbm.at[pl.ds(base, BPW)], uidx_v)
    for i in range(UIDX):
      pltpu.async_copy(uemb.at[uidx_v.at[pl.ds(i * 128, 128)]],
                       rows_v.at[pl.ds(i * 128, 128)], sem)
    for i in range(UIDX):
      pltpu.make_async_copy(uemb.at[uidx_v.at[pl.ds(i * 128, 128)]],
                            rows_v.at[pl.ds(i * 128, 128)], sem).wait()
    pltpu.sync_copy(rows_v.at[pl.ds(0, BPW)], uv_out.at[pl.ds(base, BPW)])

    pltpu.sync_copy(domain_hbm.at[pl.ds(base, BPW)], uidx_v)
    for i in range(UIDX):
      pltpu.async_copy(demb.at[uidx_v.at[pl.ds(i * 128, 128)]],
                       rows_v.at[pl.ds(i * 128, 128)], sem)
    for i in range(UIDX):
      pltpu.make_async_copy(demb.at[uidx_v.at[pl.ds(i * 128, 128)]],
                            rows_v.at[pl.ds(i * 128, 128)], sem).wait()
    pltpu.sync_copy(rows_v.at[pl.ds(0, BPW)], dv_out.at[pl.ds(base, BPW)])

    def chunk_body(c, carry):
      pltpu.sync_copy(
          title_hbm.at[pl.ds(wid * (ROWS * NCHUNK) + c * ROWS, ROWS)], tidx_v)
      for i in range(NIDX):
        pltpu.async_copy(wemb.at[tidx_v.at[pl.ds(i * 128, 128)]],
                         rows_v.at[pl.ds(i * 128, 128)], sem)
      pltpu.sync_copy(zeros_v, acc_sh.at[pl.ds(off, CH)])
      for i in range(NIDX):
        pltpu.make_async_copy(wemb.at[tidx_v.at[pl.ds(i * 128, 128)]],
                              rows_v.at[pl.ds(i * 128, 128)], sem).wait()
      for i in range(NIDX):
        pltpu.sync_copy(rows_v.at[pl.ds(i * 128, 128)],
                        acc_sh.at[dsti_v.at[i]], add=True)
      pltpu.sync_copy(acc_sh.at[pl.ds(off, CH)],
                      ts_out.at[pl.ds(base + c * CH, CH)])
      return carry

    lax.fori_loop(0, NCHUNK, chunk_body, 0)

  return k(title2d, user2d, domain2d, dstidx, word_emb, user_emb, domain_emb)


def _mlp(ts, uv, dv, w1t, w1u, w1d, b1, w2, b2):
  BLK = 1024

  def body(ts_ref, uv_ref, dv_ref, w1t_ref, w1u_ref, w1d_ref, b1_ref, w2_ref,
           b2_ref, out_ref):
    acc = lax.dot_general(ts_ref[...], w1t_ref[...], (((1,), (0,)), ((), ())),
                          preferred_element_type=jnp.float32)
    acc += lax.dot_general(uv_ref[...], w1u_ref[...], (((1,), (0,)), ((), ())),
                           preferred_element_type=jnp.float32)
    acc += lax.dot_general(dv_ref[...], w1d_ref[...], (((1,), (0,)), ((), ())),
                           preferred_element_type=jnp.float32)
    h = jnp.maximum(acc + b1_ref[...][None, :], 0.0)
    out_ref[...] = jnp.sum(h * w2_ref[...][None, :], axis=1,
                           keepdims=True) + b2_ref[0]

  grid = B // BLK
  return pl.pallas_call(
      body,
      grid=(grid,),
      in_specs=[
          pl.BlockSpec((BLK, D), lambda i: (i, 0)),
          pl.BlockSpec((BLK, D), lambda i: (i, 0)),
          pl.BlockSpec((BLK, D), lambda i: (i, 0)),
          pl.BlockSpec((D, 128), lambda i: (0, 0)),
          pl.BlockSpec((D, 128), lambda i: (0, 0)),
          pl.BlockSpec((D, 128), lambda i: (0, 0)),
          pl.BlockSpec((128,), lambda i: (0,)),
          pl.BlockSpec((128,), lambda i: (0,)),
          pl.BlockSpec(memory_space=pltpu.SMEM),
      ],
      out_specs=pl.BlockSpec((BLK, 1), lambda i: (i, 0)),
      out_shape=jax.ShapeDtypeStruct((B, 1), jnp.float32),
  )(ts, uv, dv, w1t, w1u, w1d, b1, w2, b2)


def kernel(title, user, domain, word_emb, user_emb, domain_emb, fc1_w, fc1_b,
           fc2_w, fc2_b):
  title2d = title.astype(jnp.int32).reshape(B * L)
  user2d = user.astype(jnp.int32)
  domain2d = domain.astype(jnp.int32)
  dstidx = (jnp.arange(ROWS, dtype=jnp.int32) // L).reshape(NIDX, 128)

  ts, uv, dv = _sc_gather(title2d, user2d, domain2d, dstidx, word_emb,
                          user_emb, domain_emb)

  w1t = fc1_w[:, :D].T * (1.0 / L)
  w1u = fc1_w[:, D:2 * D].T
  w1d = fc1_w[:, 2 * D:].T
  w2 = fc2_w[0]
  out = _mlp(ts, uv, dv, w1t, w1u, w1d, fc1_b, w2, fc2_b)
  return out[:, 0]


# speedup vs baseline: 5.2623x; 1.0457x over previous
import functools

import jax
import jax.numpy as jnp
from jax import lax
from jax.experimental import pallas as pl
from jax.experimental.pallas import tpu as pltpu
from jax.experimental.pallas import tpu_sc as plsc

B = 16384
L = 20
D = 64

NC = 2
NS = 16
NW = NC * NS
BPW = B // NW
CH = 64
NCHUNK = BPW // CH
ROWS = CH * L
NIDX = ROWS // 128
UIDX = BPW // 128


_MESH_KW = dict(core_axis_name="c", subcore_axis_name="s",
                num_cores=NC, num_subcores=NS)


def _sc_title(title1d, dstidx, word_emb):
  mesh = plsc.VectorSubcoreMesh(**_MESH_KW)

  @functools.partial(
      pl.kernel,
      out_type=jax.ShapeDtypeStruct((B, 2 * D), jnp.float32),
      mesh=mesh,
      scratch_types=[
          pltpu.VMEM((ROWS,), jnp.int32),
          pltpu.VMEM((NIDX, 128), jnp.int32),
          pltpu.VMEM((ROWS, D), jnp.float32),
          pltpu.VMEM((CH, D), jnp.float32),
          pltpu.VMEM_SHARED((NS * CH, D), jnp.float32),
          pltpu.SemaphoreType.DMA,
      ],
      compiler_params=pltpu.CompilerParams(use_tc_tiling_on_sc=False),
  )
  def k(title_hbm, dsti_hbm, wemb, ts_out,
        tidx_v, dsti_v, rows_v, zeros_v, acc_sh, sem):
    cid = lax.axis_index("c")
    sid = lax.axis_index("s")
    wid = sid * NC + cid
    base = wid * BPW

    pltpu.sync_copy(dsti_hbm.at[:], dsti_v)
    off = sid * CH
    for i in range(NIDX):
      for j in range(8):
        s = pl.ds(j * 16, 16)
        dsti_v[i, s] = dsti_v[i, s] + off

    zero16 = jnp.zeros((16,), jnp.float32)
    for r in range(CH):
      for j in range(D // 16):
        zeros_v[r, pl.ds(j * 16, 16)] = zero16

    def chunk_body(c, carry):
      pltpu.sync_copy(
          title_hbm.at[pl.ds(wid * (ROWS * NCHUNK) + c * ROWS, ROWS)], tidx_v)
      for i in range(NIDX):
        pltpu.async_copy(wemb.at[tidx_v.at[pl.ds(i * 128, 128)]],
                         rows_v.at[pl.ds(i * 128, 128)], sem)
      pltpu.sync_copy(zeros_v, acc_sh.at[pl.ds(off, CH)])
      for i in range(NIDX):
        pltpu.make_async_copy(wemb.at[tidx_v.at[pl.ds(i * 128, 128)]],
                              rows_v.at[pl.ds(i * 128, 128)], sem).wait()
      for i in range(NIDX):
        pltpu.sync_copy(rows_v.at[pl.ds(i * 128, 128)],
                        acc_sh.at[dsti_v.at[i]], add=True)
      pltpu.sync_copy(acc_sh.at[pl.ds(off, CH)],
                      ts_out.at[pl.ds(base + c * CH, CH), pl.ds(0, D)])
      return carry

    lax.fori_loop(0, NCHUNK, chunk_body, 0)

  return k(title1d, dstidx, word_emb)


def _sc_user_domain(user1d, domain1d, user_emb, domain_emb):
  mesh = plsc.VectorSubcoreMesh(**_MESH_KW)

  @functools.partial(
      pl.kernel,
      out_type=(
          jax.ShapeDtypeStruct((B, 2 * D), jnp.float32),
          jax.ShapeDtypeStruct((B, 2 * D), jnp.float32),
      ),
      mesh=mesh,
      scratch_types=[
          pltpu.VMEM((BPW,), jnp.int32),
          pltpu.VMEM((BPW, D), jnp.float32),
          pltpu.SemaphoreType.DMA,
      ],
      compiler_params=pltpu.CompilerParams(use_tc_tiling_on_sc=False),
  )
  def k(user_hbm, domain_hbm, uemb, demb, uv_out, dv_out,
        uidx_v, rows_v, sem):
    cid = lax.axis_index("c")
    sid = lax.axis_index("s")
    wid = sid * NC + cid
    base = wid * BPW

    for idx_hbm, emb, out in ((user_hbm, uemb, uv_out),
                              (domain_hbm, demb, dv_out)):
      pltpu.sync_copy(idx_hbm.at[pl.ds(base, BPW)], uidx_v)
      for i in range(UIDX):
        pltpu.async_copy(emb.at[uidx_v.at[pl.ds(i * 128, 128)]],
                         rows_v.at[pl.ds(i * 128, 128)], sem)
      for i in range(UIDX):
        pltpu.make_async_copy(emb.at[uidx_v.at[pl.ds(i * 128, 128)]],
                              rows_v.at[pl.ds(i * 128, 128)], sem).wait()
      pltpu.sync_copy(rows_v, out.at[pl.ds(base, BPW), pl.ds(0, D)])

  return k(user1d, domain1d, user_emb, domain_emb)


def _mlp(ts, uv, dv, w1t, w1u, w1d, b1, w2, b2):
  BLK = 1024

  def body(ts_ref, uv_ref, dv_ref, w1t_ref, w1u_ref, w1d_ref, b1_ref, w2_ref,
           b2_ref, out_ref):
    s = pl.ds(0, D)
    acc = lax.dot_general(ts_ref[:, s], w1t_ref[...],
                          (((1,), (0,)), ((), ())),
                          preferred_element_type=jnp.float32)
    acc += lax.dot_general(uv_ref[:, s], w1u_ref[...],
                           (((1,), (0,)), ((), ())),
                           preferred_element_type=jnp.float32)
    acc += lax.dot_general(dv_ref[:, s], w1d_ref[...],
                           (((1,), (0,)), ((), ())),
                           preferred_element_type=jnp.float32)
    h = jnp.maximum(acc + b1_ref[...][None, :], 0.0)
    out_ref[...] = jnp.sum(h * w2_ref[...][None, :], axis=1,
                           keepdims=True) + b2_ref[0]

  grid = B // BLK
  return pl.pallas_call(
      body,
      grid=(grid,),
      in_specs=[
          pl.BlockSpec((BLK, 2 * D), lambda i: (i, 0)),
          pl.BlockSpec((BLK, 2 * D), lambda i: (i, 0)),
          pl.BlockSpec((BLK, 2 * D), lambda i: (i, 0)),
          pl.BlockSpec((D, 128), lambda i: (0, 0)),
          pl.BlockSpec((D, 128), lambda i: (0, 0)),
          pl.BlockSpec((D, 128), lambda i: (0, 0)),
          pl.BlockSpec((128,), lambda i: (0,)),
          pl.BlockSpec((128,), lambda i: (0,)),
          pl.BlockSpec(memory_space=pltpu.SMEM),
      ],
      out_specs=pl.BlockSpec((BLK, 1), lambda i: (i, 0)),
      out_shape=jax.ShapeDtypeStruct((B, 1), jnp.float32),
  )(ts, uv, dv, w1t, w1u, w1d, b1, w2, b2)


def kernel(title, user, domain, word_emb, user_emb, domain_emb, fc1_w, fc1_b,
           fc2_w, fc2_b):
  title1d = title.astype(jnp.int32).reshape(B * L)
  user1d = user.astype(jnp.int32)
  domain1d = domain.astype(jnp.int32)
  dstidx = (jnp.arange(ROWS, dtype=jnp.int32) // L).reshape(NIDX, 128)

  ts = _sc_title(title1d, dstidx, word_emb)
  uv, dv = _sc_user_domain(user1d, domain1d, user_emb, domain_emb)

  w1t = fc1_w[:, :D].T * (1.0 / L)
  w1u = fc1_w[:, D:2 * D].T
  w1d = fc1_w[:, 2 * D:].T
  w2 = fc2_w[0]
  out = _mlp(ts, uv, dv, w1t, w1u, w1d, fc1_b, w2, fc2_b)
  return out[:, 0]
